# Initial kernel scaffold; baseline (speedup 1.0000x reference)
import jax
import jax.numpy as jnp
from jax.experimental import pallas as pl


def kernel(input, target):
    raise NotImplementedError("write your pallas kernel here")


# speedup vs baseline: 462.6506x; 462.6506x over previous
import functools

import jax
import jax.numpy as jnp
from jax.experimental import pallas as pl
from jax.experimental.pallas import tpu as pltpu

_MAX_TRIALS = 10

_RW = [1.0]
for _i in range(1, _MAX_TRIALS + 1):
    _RW.append(_RW[-1] + 1.0 / _i + 1.0)
_W1, _W2, _W3, _W5, _W10 = _RW[1], _RW[2], _RW[3], _RW[5], _RW[10]

_ROWS = 8


def _body(x_ref, t_ref, out_ref):
    i = pl.program_id(0)
    x = x_ref[...]
    t = t_ref[...]
    lsz = x.shape[1]
    neg = t == 0
    z = jnp.where(neg, x, -jnp.inf)
    nneg = jnp.sum(neg.astype(jnp.float32), axis=1, keepdims=True)
    cnt = jnp.zeros(x.shape, jnp.float32)
    for k in range(lsz):
        cnt += (z[:, k : k + 1] >= x).astype(jnp.float32)
    p = cnt / jnp.maximum(nneg, 1.0)
    q = 1.0 - p
    q2 = q * q
    q3 = q2 * q
    q4 = q2 * q2
    q5 = q4 * q
    el = p * (_W10 + q * _W5 + q2 * _W3 + (q3 + q4) * _W2) + q5 * _W1
    el = jnp.where(t == 1, el, 0.0)
    s = jnp.float32(lsz) - jnp.sum(x * (2.0 * t.astype(jnp.float32) - 1.0), axis=1)
    partial = jnp.sum(s * jnp.sum(el, axis=1))

    @pl.when(i == 0)
    def _():
        out_ref[0, 0] = 0.0

    out_ref[0, 0] += partial


@jax.jit
def kernel(input, target):
    b, lsz = input.shape
    out = pl.pallas_call(
        _body,
        grid=(b // _ROWS,),
        in_specs=[
            pl.BlockSpec((_ROWS, lsz), lambda i: (i, 0)),
            pl.BlockSpec((_ROWS, lsz), lambda i: (i, 0)),
        ],
        out_specs=pl.BlockSpec((1, 1), lambda i: (0, 0), memory_space=pltpu.SMEM),
        out_shape=jax.ShapeDtypeStruct((1, 1), jnp.float32),
    )(input, target)
    return out.reshape(())
